# baseline (speedup 1.0000x reference)
import jax
import jax.numpy as jnp
from jax import lax
from jax.experimental import pallas as pl
from jax.experimental.pallas import tpu as pltpu
from jax.experimental.pallas import tpu_sc as plsc

_QR_RATIO = 64
_EMB_DIM = 64
_LANES = 16
_NC = 2
_NS = 16
_NW = _NC * _NS

_B = 16384 * 26
_PW = _B // _NW
_C = 512
_NCH = _PW // _C
_GSZ = 128
_NG = _C // _GSZ


def _body(x_hbm, embq_hbm, embr_hbm, out_hbm,
          rloc, idx0, idx1, qi0, qi1, ri0, ri1, rq0, rq1,
          semi0, semi1, semq0, semq1, semo0, semo1):
    wid = lax.axis_index("s") * _NC + lax.axis_index("c")
    base_w = wid * _PW
    idx, qi, ri = [idx0, idx1], [qi0, qi1], [ri0, ri1]
    rq = [rq0, rq1]
    semi, semq, semo = [semi0, semi1], [semq0, semq1], [semo0, semo1]

    def idx_copy(ch, b):
        return pltpu.make_async_copy(
            x_hbm.at[pl.ds(base_w + ch * _C, _C)], idx[b], semi[b])

    def out_copy(ch, b):
        return pltpu.make_async_copy(
            rq[b], out_hbm.at[pl.ds(base_w + ch * _C, _C)], semo[b])

    def gather_copies(b):
        return [pltpu.make_async_copy(
                    embq_hbm.at[qi[b].at[s]],
                    rq[b].at[pl.ds(s * _GSZ, _GSZ)], semq[b])
                for s in range(_NG)]

    def compute_qr(b):
        for i in range(_C // _LANES):
            v = idx[b][pl.ds(i * _LANES, _LANES)]
            g = i // (_GSZ // _LANES)
            o = (i % (_GSZ // _LANES)) * _LANES
            qi[b][g, pl.ds(o, _LANES)] = v >> 6
            ri[b][pl.ds(i * _LANES, _LANES)] = v & (_QR_RATIO - 1)

    def add_rows(b):
        def group(g, c):
            rvec = ri[b][pl.ds(g * _LANES, _LANES)]
            rowid = g * _LANES + lax.iota(jnp.int32, _LANES)
            for col in range(_EMB_DIM):
                cvec = jnp.full((_LANES,), col, jnp.int32)
                rv = plsc.load_gather(rloc, [rvec, cvec])
                plsc.addupdate_scatter(rq[b], [rowid, cvec], rv)
            return c
        lax.fori_loop(0, _C // _LANES, group, 0)

    idx_copy(0, 0).start()
    idx_copy(1, 1).start()
    pltpu.sync_copy(embr_hbm, rloc)
    idx_copy(0, 0).wait()
    compute_qr(0)
    for cp in gather_copies(0):
        cp.start()

    def iter_body(p, carry):
        for b in (0, 1):
            ch = p * 2 + b
            nb = 1 - b

            @pl.when(ch + 1 < _NCH)
            def _prep():
                idx_copy(ch + 1, nb).wait()
                compute_qr(nb)

                @pl.when(ch + 2 < _NCH)
                def _pf():
                    idx_copy(ch + 2, b).start()

                @pl.when(ch >= 1)
                def _wo():
                    out_copy(ch - 1, nb).wait()
                for cp in gather_copies(nb):
                    cp.start()

            for cp in gather_copies(b):
                cp.wait()
            add_rows(b)
            out_copy(ch, b).start()
        return carry

    lax.fori_loop(0, _NCH // 2, iter_body, 0)
    out_copy(_NCH - 2, 0).wait()
    out_copy(_NCH - 1, 1).wait()


@jax.jit
def _qr_embed(x_flat, embedding_q, embedding_r):
    mesh = plsc.VectorSubcoreMesh(
        core_axis_name="c", subcore_axis_name="s",
        num_cores=_NC, num_subcores=_NS)
    return pl.kernel(
        _body,
        out_type=jax.ShapeDtypeStruct((_B, _EMB_DIM), jnp.float32),
        mesh=mesh,
        scratch_types=[
            pltpu.VMEM((_QR_RATIO, _EMB_DIM), jnp.float32),
            pltpu.VMEM((_C,), jnp.int32),
            pltpu.VMEM((_C,), jnp.int32),
            pltpu.VMEM((_NG, _GSZ), jnp.int32),
            pltpu.VMEM((_NG, _GSZ), jnp.int32),
            pltpu.VMEM((_C,), jnp.int32),
            pltpu.VMEM((_C,), jnp.int32),
            pltpu.VMEM((_C, _EMB_DIM), jnp.float32),
            pltpu.VMEM((_C, _EMB_DIM), jnp.float32),
            pltpu.SemaphoreType.DMA,
            pltpu.SemaphoreType.DMA,
            pltpu.SemaphoreType.DMA,
            pltpu.SemaphoreType.DMA,
            pltpu.SemaphoreType.DMA,
            pltpu.SemaphoreType.DMA,
        ],
        compiler_params=pltpu.CompilerParams(
            use_tc_tiling_on_sc=False, needs_layout_passes=False),
    )(x_flat, embedding_q, embedding_r)


def kernel(x, embedding_q, embedding_r):
    b, f = x.shape
    x_flat = x.reshape(-1).astype(jnp.int32)
    out = _qr_embed(x_flat, embedding_q, embedding_r)
    return out.reshape(b, f, _EMB_DIM)


# speedup vs baseline: 3.4181x; 3.4181x over previous
import jax
import jax.numpy as jnp
from jax import lax
from jax.experimental import pallas as pl
from jax.experimental.pallas import tpu as pltpu
from jax.experimental.pallas import tpu_sc as plsc

_QR_RATIO = 64
_EMB_DIM = 64
_LANES = 16
_NC = 2
_NS = 16
_NW = _NC * _NS
_QROWS = 15627

_B = 16384 * 26
_PW = _B // _NW
_C = 256
_NCH = _PW // _C
_GSZ = 128
_NG = _C // _GSZ


def _body(x_hbm, embq_hbm, embr_hbm, out_hbm,
          spq, spr,
          idx0, idx1, qi0, qi1, ri0, ri1, rq0, rq1, rr0, rr1,
          semi0, semi1, semq0, semq1, semo0, semo1):
    wid = lax.axis_index("s") * _NC + lax.axis_index("c")
    base_w = wid * _PW
    idx, qi, ri = [idx0, idx1], [qi0, qi1], [ri0, ri1]
    rq, rr = [rq0, rq1], [rr0, rr1]
    semi, semq, semo = [semi0, semi1], [semq0, semq1], [semo0, semo1]

    def idx_copy(ch, b):
        return pltpu.make_async_copy(
            x_hbm.at[pl.ds(base_w + ch * _C, _C)], idx[b], semi[b])

    def out_copy(ch, b):
        return pltpu.make_async_copy(
            rq[b], out_hbm.at[pl.ds(base_w + ch * _C, _C)], semo[b])

    def gather_copies(b):
        cps = []
        for s in range(_NG):
            dst = pl.ds(s * _GSZ, _GSZ)
            cps.append(pltpu.make_async_copy(
                spq.at[qi[b].at[s]], rq[b].at[dst], semq[b]))
            cps.append(pltpu.make_async_copy(
                spr.at[ri[b].at[s]], rr[b].at[dst], semq[b]))
        return cps

    def compute_qr(b):
        for i in range(_C // _LANES):
            v = idx[b][pl.ds(i * _LANES, _LANES)]
            g = i // (_GSZ // _LANES)
            o = (i % (_GSZ // _LANES)) * _LANES
            qi[b][g, pl.ds(o, _LANES)] = v >> 6
            ri[b][g, pl.ds(o, _LANES)] = v & (_QR_RATIO - 1)

    def add_rows(b):
        def body4(k, c):
            for u in range(4):
                row = k * 4 + u
                for j in range(_EMB_DIM // _LANES):
                    blk = pl.ds(j * _LANES, _LANES)
                    plsc.addupdate(rq[b].at[row, blk], rr[b][row, blk])
            return c
        lax.fori_loop(0, _C // 4, body4, 0)

    idx_copy(0, 0).start()
    idx_copy(1, 1).start()

    @pl.when(lax.axis_index("s") == 0)
    def _stage():
        pltpu.sync_copy(embq_hbm, spq)
        pltpu.sync_copy(embr_hbm, spr)
    plsc.subcore_barrier()

    idx_copy(0, 0).wait()
    compute_qr(0)
    for cp in gather_copies(0):
        cp.start()

    def iter_body(p, carry):
        for b in (0, 1):
            ch = p * 2 + b
            nb = 1 - b

            @pl.when(ch + 1 < _NCH)
            def _prep():
                idx_copy(ch + 1, nb).wait()
                compute_qr(nb)

                @pl.when(ch + 2 < _NCH)
                def _pf():
                    idx_copy(ch + 2, b).start()

                @pl.when(ch >= 1)
                def _wo():
                    out_copy(ch - 1, nb).wait()
                for cp in gather_copies(nb):
                    cp.start()

            for cp in gather_copies(b):
                cp.wait()
            add_rows(b)
            out_copy(ch, b).start()
        return carry

    lax.fori_loop(0, _NCH // 2, iter_body, 0)
    out_copy(_NCH - 2, 0).wait()
    out_copy(_NCH - 1, 1).wait()


@jax.jit
def _qr_embed(x_flat, embedding_q, embedding_r):
    mesh = plsc.VectorSubcoreMesh(
        core_axis_name="c", subcore_axis_name="s",
        num_cores=_NC, num_subcores=_NS)
    return pl.kernel(
        _body,
        out_type=jax.ShapeDtypeStruct((_B, _EMB_DIM), jnp.float32),
        mesh=mesh,
        scratch_types=[
            pltpu.VMEM_SHARED((_QROWS, _EMB_DIM), jnp.float32),
            pltpu.VMEM_SHARED((_QR_RATIO, _EMB_DIM), jnp.float32),
            pltpu.VMEM((_C,), jnp.int32),
            pltpu.VMEM((_C,), jnp.int32),
            pltpu.VMEM((_NG, _GSZ), jnp.int32),
            pltpu.VMEM((_NG, _GSZ), jnp.int32),
            pltpu.VMEM((_NG, _GSZ), jnp.int32),
            pltpu.VMEM((_NG, _GSZ), jnp.int32),
            pltpu.VMEM((_C, _EMB_DIM), jnp.float32),
            pltpu.VMEM((_C, _EMB_DIM), jnp.float32),
            pltpu.VMEM((_C, _EMB_DIM), jnp.float32),
            pltpu.VMEM((_C, _EMB_DIM), jnp.float32),
            pltpu.SemaphoreType.DMA,
            pltpu.SemaphoreType.DMA,
            pltpu.SemaphoreType.DMA,
            pltpu.SemaphoreType.DMA,
            pltpu.SemaphoreType.DMA,
            pltpu.SemaphoreType.DMA,
        ],
        compiler_params=pltpu.CompilerParams(
            use_tc_tiling_on_sc=False, needs_layout_passes=False),
    )(x_flat, embedding_q, embedding_r)


def kernel(x, embedding_q, embedding_r):
    b, f = x.shape
    x_flat = x.reshape(-1).astype(jnp.int32)
    out = _qr_embed(x_flat, embedding_q, embedding_r)
    return out.reshape(b, f, _EMB_DIM)
